# Initial kernel scaffold; baseline (speedup 1.0000x reference)
import jax
import jax.numpy as jnp
from jax.experimental import pallas as pl


def kernel(step, lprobs, scores, mask):
    raise NotImplementedError("write your pallas kernel here")


# speedup vs baseline: 6.6120x; 6.6120x over previous
import functools

import jax
import jax.numpy as jnp
from jax import lax
from jax.experimental import pallas as pl
from jax.experimental.pallas import tpu as pltpu
from jax.experimental.pallas import tpu_sc as plsc

BSZ = 32
NBEAM = 8
VOCAB = 100000
K = 16
LANES = 16
CAND_MULT = 2

CHUNK = 20000
NCHUNKS = VOCAB // CHUNK
G = 5
GSZ = G * LANES
NGROUPS = CHUNK // GSZ
CAP = 2048

NEG_INF = float("-inf")
IMAX = 2**31 - 1


def _sel16(cval, cidx, nvec, lane):
    sval = jnp.full((LANES,), NEG_INF, jnp.float32)
    sidx = jnp.zeros((LANES,), jnp.int32)
    for r in range(K):
        def scan_body(t, carry):
            bv, bi = carry
            v = cval[pl.ds(t * LANES, LANES)]
            i = cidx[pl.ds(t * LANES, LANES)]
            better = (v > bv) | ((v == bv) & (i < bi))
            return jnp.where(better, v, bv), jnp.where(better, i, bi)

        bv, bi = lax.fori_loop(
            0, nvec, scan_body,
            (jnp.full((LANES,), NEG_INF, jnp.float32),
             jnp.full((LANES,), IMAX, jnp.int32)))
        mval = jnp.max(bv, axis=0)
        midx = jnp.min(jnp.where(bv == mval, bi, IMAX), axis=0)
        hit = lane == r
        sval = jnp.where(hit, mval, sval)
        sidx = jnp.where(hit, midx, sidx)

        def kill_body(t, _):
            v = cval[pl.ds(t * LANES, LANES)]
            i = cidx[pl.ds(t * LANES, LANES)]
            cval[pl.ds(t * LANES, LANES)] = jnp.where(i == midx, NEG_INF, v)
            return 0

        lax.fori_loop(0, nvec, kill_body, 0)
    return sval, sidx


def _make_kernel():
    mesh = plsc.VectorSubcoreMesh(core_axis_name="c", subcore_axis_name="s")

    @functools.partial(
        pl.kernel,
        mesh=mesh,
        compiler_params=pltpu.CompilerParams(needs_layout_passes=False),
        out_type=[
            jax.ShapeDtypeStruct((BSZ, K), jnp.float32),
            jax.ShapeDtypeStruct((BSZ, K), jnp.int32),
            jax.ShapeDtypeStruct((BSZ, K), jnp.int32),
        ],
        scratch_types=[
            pltpu.VMEM((CHUNK,), jnp.float32),
            pltpu.VMEM((CAP,), jnp.float32),
            pltpu.VMEM((CAP,), jnp.int32),
            pltpu.VMEM((BSZ * LANES,), jnp.float32),
            pltpu.VMEM((BSZ * LANES,), jnp.int32),
            pltpu.VMEM((K,), jnp.float32),
            pltpu.VMEM((K,), jnp.int32),
            pltpu.VMEM((K,), jnp.int32),
        ],
    )
    def topk_kernel(lp_hbm, bias_hbm, mask_hbm, val_out, idx_out, beam_out,
                    chunk_v, cval, cidx, bias_v, mask_v, sv, si, sb):
        wid = lax.axis_index("s") * 2 + lax.axis_index("c")
        row = wid
        lane = lax.iota(jnp.int32, LANES)

        pltpu.sync_copy(bias_hbm, bias_v)
        pltpu.sync_copy(mask_hbm, mask_v)
        bias_vec = bias_v[pl.ds(row * LANES, LANES)]
        mask_vec = mask_v[pl.ds(row * LANES, LANES)]

        def keep(tc):
            return tc

        def compact(tc):
            _, cnt0 = tc
            w_val, w_idx = _sel16(cval, cidx, cnt0 >> 4, lane)
            cval[pl.ds(0, LANES)] = w_val
            cidx[pl.ds(0, LANES)] = w_idx
            return jnp.min(w_val, axis=0), jnp.int32(K)

        def beam_body(beam, tc):
            bsel = jnp.full((LANES,), beam, jnp.int32)
            bias_spl = bias_vec.at[bsel].get(mode="promise_in_bounds")
            mask_spl = mask_vec.at[bsel].get(mode="promise_in_bounds")
            idx0 = beam * VOCAB

            def masked_case(tc1):
                thr1, cnt1 = lax.cond(tc1[1] > CAP - LANES, compact, keep, tc1)

                def app(tc2):
                    thr2, cnt2 = tc2
                    cval[pl.ds(cnt2, LANES)] = bias_spl
                    cidx[pl.ds(cnt2, LANES)] = idx0 + lane
                    return thr2, cnt2 + LANES

                return lax.cond(jnp.any(bias_spl > thr1), app, keep,
                                (thr1, cnt1))

            def stream_case(tc1):
                def chunk_body(c, tc2):
                    off = row * (NBEAM * VOCAB) + idx0 + c * CHUNK
                    pltpu.sync_copy(lp_hbm.at[pl.ds(off, CHUNK)], chunk_v)
                    idx_base = idx0 + c * CHUNK

                    def group_body(g, tc3):
                        tc3 = lax.cond(tc3[1] > CAP - GSZ, compact, keep, tc3)
                        thr3, cnt3 = tc3
                        base = g * G
                        anym = None
                        for u in range(G):
                            v = chunk_v[pl.ds((base + u) * LANES, LANES)]
                            m = (v + bias_spl) > thr3
                            anym = m if anym is None else (anym | m)

                        def app(tc4):
                            thr4, cnt4 = tc4
                            for u in range(G):
                                v = chunk_v[pl.ds((base + u) * LANES, LANES)]
                                val = v + bias_spl

                                def a2(cnt5, val=val, u=u):
                                    cval[pl.ds(cnt5, LANES)] = val
                                    cidx[pl.ds(cnt5, LANES)] = (
                                        idx_base + (base + u) * LANES + lane)
                                    return cnt5 + LANES

                                cnt4 = lax.cond(jnp.any(val > thr4), a2,
                                                lambda c5: c5, cnt4)
                            return thr4, cnt4

                        return lax.cond(jnp.any(anym), app, keep,
                                        (thr3, cnt3))

                    return lax.fori_loop(0, NGROUPS, group_body, tc2)

                return lax.fori_loop(0, NCHUNKS, chunk_body, tc1)

            return lax.cond(jnp.any(mask_spl == 0), masked_case,
                            stream_case, tc)

        thr, cnt = lax.fori_loop(0, NBEAM, beam_body,
                                 (jnp.float32(NEG_INF), jnp.int32(0)))

        w_val, w_idx = _sel16(cval, cidx, cnt >> 4, lane)
        w_beam = w_idx // VOCAB
        w_vocab = w_idx - w_beam * VOCAB
        sv[...] = w_val
        si[...] = w_vocab
        sb[...] = w_beam
        pltpu.sync_copy(sv, val_out.at[row])
        pltpu.sync_copy(si, idx_out.at[row])
        pltpu.sync_copy(sb, beam_out.at[row])

    return topk_kernel


_TOPK = _make_kernel()


def kernel(step, lprobs, scores, mask):
    bsz, beam_size, vocab_size = lprobs.shape
    bias = lax.dynamic_index_in_dim(scores, step - 1, axis=2, keepdims=False)
    bias_p = jnp.pad(bias.astype(jnp.float32),
                     ((0, 0), (0, LANES - beam_size))).reshape(-1)
    mask_p = jnp.pad(mask.astype(jnp.int32),
                     ((0, 0), (0, LANES - beam_size)),
                     constant_values=1).reshape(-1)
    lp_flat = lprobs.reshape(-1)
    vals, vidx, beams = _TOPK(lp_flat, bias_p, mask_p)
    return vals, vidx, beams
